# baseline (speedup 1.0000x reference)
import functools

import jax
import jax.numpy as jnp
from jax import lax
from jax.experimental import pallas as pl
from jax.experimental.pallas import tpu as pltpu
from jax.experimental.pallas import tpu_sc as plsc

_B = 8
_N = 2048
_TOTAL = 8192
_E = 262144
_D = 512
_BN = 512

_NTILE = 16
_EP = _E // _NTILE
_NG = _EP // 16
_NROWS = 128
_CHUNK = (_N // 8) * _N
_TRASH = _CHUNK
_HUGE = 1 << 30


def _dense_body(batch_ref, x_ref, out_ref, cum_smem, buf0, buf1, iota_v,
                sem0, sem1):
    b = pl.program_id(0)
    j = pl.program_id(1)
    nj = _N // _BN
    step = b * nj + j
    bufs = (buf0, buf1)
    sems = (sem0, sem1)
    L = _BN * _D

    def src_off(k):
        bb = k // nj
        jj = k % nj
        return (cum_smem[bb] + jj * _BN) * _D

    def start(k, buf, sem):
        pltpu.make_async_copy(x_ref.at[pl.ds(src_off(k), L)], buf, sem).start()

    def wait(k, buf, sem):
        pltpu.make_async_copy(x_ref.at[pl.ds(src_off(k), L)], buf, sem).wait()

    @pl.when(step == 0)
    def _():
        bi = batch_ref[...]
        for k in range(_B + 1):
            cum_smem[k] = jnp.sum((bi < k).astype(jnp.int32))
        start(0, buf0, sem0)
        start(1, buf1, sem1)
        iota_v[...] = lax.broadcasted_iota(jnp.int32, (L,), 0)

    parity = step % 2
    nsteps = _B * nj

    for par in range(2):
        @pl.when(parity == par)
        def _():
            buf, sem = bufs[par], sems[par]
            wait(step, buf, sem)
            thr = (cum_smem[b + 1] - cum_smem[b] - j * _BN) * _D

            @pl.when(thr >= L)
            def _():
                out_ref[...] = buf[...]

            @pl.when(thr <= 0)
            def _():
                out_ref[...] = jnp.zeros((L,), jnp.float32)

            @pl.when((thr > 0) & (thr < L))
            def _():
                out_ref[...] = jnp.where(iota_v[...] < thr, buf[...], 0.0)

            @pl.when(step + 2 < nsteps)
            def _():
                start(step + 2, buf, sem)


def _dense_x(x, batch_idx):
    x_pad = jnp.concatenate(
        [x.reshape(-1), jnp.zeros((_N * _D,), x.dtype)], axis=0)
    batch2d = batch_idx.reshape(_TOTAL // 128, 128)
    out = pl.pallas_call(
        _dense_body,
        grid=(_B, _N // _BN),
        in_specs=[
            pl.BlockSpec((_TOTAL // 128, 128), lambda b, j: (0, 0)),
            pl.BlockSpec(memory_space=pltpu.MemorySpace.HBM),
        ],
        out_specs=pl.BlockSpec((_BN * _D,), lambda b, j: (b * (_N // _BN) + j,)),
        out_shape=jax.ShapeDtypeStruct((_B * _N * _D,), jnp.float32),
        scratch_shapes=[
            pltpu.SMEM((_B + 1,), jnp.int32),
            pltpu.VMEM((_BN * _D,), jnp.float32),
            pltpu.VMEM((_BN * _D,), jnp.float32),
            pltpu.VMEM((_BN * _D,), jnp.int32),
            pltpu.SemaphoreType.DMA,
            pltpu.SemaphoreType.DMA,
        ],
    )(batch2d, x_pad)
    return out.reshape(_B, _N, _D)


_MESH = plsc.VectorSubcoreMesh(core_axis_name="c", subcore_axis_name="s")


@functools.partial(
    pl.kernel,
    out_type=jax.ShapeDtypeStruct((_N, _N), jnp.float32),
    mesh=_MESH,
    compiler_params=pltpu.CompilerParams(needs_layout_passes=False),
    scratch_types=[
        pltpu.VMEM((_EP,), jnp.int32),
        pltpu.VMEM((_EP,), jnp.int32),
        pltpu.VMEM((_EP + 16,), jnp.int32),
        pltpu.VMEM((_TOTAL,), jnp.int32),
        pltpu.VMEM((16,), jnp.int32),
        pltpu.VMEM((_NROWS + 1, 128), jnp.int32),
        pltpu.VMEM((128,), jnp.float32),
        pltpu.VMEM((16384,), jnp.float32),
        pltpu.VMEM_SHARED((_CHUNK + 64,), jnp.float32),
        pltpu.SemaphoreType.DMA,
    ],
)
def _adj_kernel(edge_ref, batch_ref, adj_ref,
                src_v, dst_v, vlist, batch_v, cumtab, idx2d, ones_v, zeros_v,
                accum, dsem):
    cid = lax.axis_index("c")
    sid = lax.axis_index("s")
    base_e = sid * _EP
    pltpu.sync_copy(edge_ref.at[0, pl.ds(base_e, _EP)], src_v)
    pltpu.sync_copy(edge_ref.at[1, pl.ds(base_e, _EP)], dst_v)
    pltpu.sync_copy(batch_ref, batch_v)

    zero16f = jnp.zeros((16,), jnp.float32)

    def fill_z(i, c):
        zeros_v[pl.ds(i * 16, 16)] = zero16f
        return c

    lax.fori_loop(0, 16384 // 16, fill_z, 0)
    one16f = jnp.ones((16,), jnp.float32)
    for i in range(128 // 16):
        ones_v[pl.ds(i * 16, 16)] = one16f

    one_b = jnp.full((16,), 1, jnp.int32)
    zero_b = jnp.full((16,), 0, jnp.int32)
    kconsts = [jnp.full((16,), k + 1, jnp.int32) for k in range(7)]

    def cum_body(g, accs):
        v = batch_v[pl.ds(g * 16, 16)]
        return tuple(a + jnp.where(v < kconsts[k], one_b, zero_b)
                     for k, a in enumerate(accs))

    accs = lax.fori_loop(0, _TOTAL // 16, cum_body,
                         tuple(jnp.zeros((16,), jnp.int32) for _ in range(7)))

    lanes = lax.broadcasted_iota(jnp.int32, (16,), 0)
    gdn = lax.GatherDimensionNumbers(
        offset_dims=(), collapsed_slice_dims=(0,), start_index_map=(0,))

    def gat(v, idx):
        return lax.gather(v, idx[:, None], gdn, (1,),
                          mode=lax.GatherScatterMode.PROMISE_IN_BOUNDS)

    def lane_sum(v):
        for sh in (8, 4, 2, 1):
            v = v + gat(v, jnp.bitwise_xor(lanes, jnp.full((16,), sh,
                                                           jnp.int32)))
        return v

    n_b = jnp.full((16,), _N, jnp.int32)
    cumb = [lane_sum(a) for a in accs]
    c1n = jnp.minimum(cumb[0], n_b)

    ctab = zero_b
    for k in range(7):
        ctab = jnp.where(lanes == kconsts[k], cumb[k], ctab)
    cumtab[pl.ds(0, 16)] = ctab

    pidx = [jnp.maximum(lanes - jnp.full((16,), sh, jnp.int32), zero_b)
            for sh in (1, 2, 4, 8)]
    pmask = [lanes >= jnp.full((16,), sh, jnp.int32) for sh in (1, 2, 4, 8)]

    def prefix16(v):
        for i in range(4):
            v = v + jnp.where(pmask[i], gat(v, pidx[i]), zero_b)
        return v

    huge_b = jnp.full((16,), _HUGE, jnp.int32)
    c11_b = jnp.full((16,), 11, jnp.int32)
    c3s_b = jnp.full((16,), 3, jnp.int32)
    m7_b = jnp.full((16,), 7, jnp.int32)
    m11_b = jnp.full((16,), 2047, jnp.int32)
    c7s_b = jnp.full((16,), 7, jnp.int32)
    m127_b = jnp.full((16,), 127, jnp.int32)
    lim_b = jnp.full((16,), _N // 8, jnp.int32)
    trash_b = jnp.full((16,), _TRASH, jnp.int32)

    def build(g, off_b):
        s = src_v[pl.ds(g * 16, 16)]
        t = dst_v[pl.ds(g * 16, 16)]
        bdst = plsc.load_gather(batch_v, [t])
        csel = plsc.load_gather(cumtab, [bdst])
        i2 = t - csel
        m = (s < c1n) & (i2 < n_b)
        gidx = lax.shift_left(s, c11_b) + i2
        m01 = jnp.where(m, one_b, zero_b)
        pref = prefix16(m01)
        pos = off_b + pref - m01
        plsc.store_scatter(vlist, [pos], gidx, mask=m)
        return off_b + plsc.all_reduce_population_count(m)

    off_b = lax.fori_loop(0, _NG, build, zero_b)
    nv = off_b[0]
    vlist[pl.ds(nv, 16)] = huge_b
    ngv = (nv + 15) // 16

    tile_words = _CHUNK // _NTILE

    for half in range(4):
        p = cid * 4 + half

        for q in range(2):
            pltpu.async_copy(
                zeros_v, accum.at[pl.ds(sid * tile_words + q * 16384, 16384)],
                dsem)

        @pl.when(sid == 0)
        def _():
            pltpu.sync_copy(zeros_v.at[pl.ds(0, 64)],
                            accum.at[pl.ds(_CHUNK, 64)])

        for q in range(2):
            pltpu.make_async_copy(
                zeros_v, accum.at[pl.ds(sid * tile_words + q * 16384, 16384)],
                dsem).wait()

        plsc.subcore_barrier()

        pb = jnp.full((16,), p, jnp.int32)

        def mkloc(g, off2):
            gv = vlist[pl.ds(g * 16, 16)]
            i1 = lax.shift_right_logical(gv, c11_b)
            i2 = jnp.bitwise_and(gv, m11_b)
            r = lax.shift_right_logical(i1, c3s_b)
            m = (jnp.bitwise_and(i1, m7_b) == pb) & (r < lim_b)
            loc = jnp.bitwise_or(lax.shift_left(r, c11_b), i2)
            m01 = jnp.where(m, one_b, zero_b)
            pref = prefix16(m01)
            pos = off2 + pref - m01
            plsc.store_scatter(
                idx2d,
                [lax.shift_right_logical(pos, c7s_b),
                 jnp.bitwise_and(pos, m127_b)],
                loc, mask=m)
            return off2 + plsc.all_reduce_population_count(m)

        off2_b = lax.fori_loop(0, ngv, mkloc, zero_b)
        cnt2 = off2_b[0]

        for k in range(8):
            pos = off2_b + jnp.full((16,), k * 16, jnp.int32) + lanes
            plsc.store_scatter(
                idx2d,
                [lax.shift_right_logical(pos, c7s_b),
                 jnp.bitwise_and(pos, m127_b)],
                trash_b)

        ns = (cnt2 + 127) // 128

        def scat(j, c):
            pltpu.sync_copy(ones_v, accum.at[idx2d.at[j]], add=True)
            return c

        for t in range(_NTILE):
            @pl.when(sid == t)
            def _():
                lax.fori_loop(0, ns, scat, 0)

            plsc.subcore_barrier()
        plsc.subcore_barrier()

        def cout(q, c):
            r = sid * 16 + q
            pltpu.async_copy(accum.at[pl.ds(r * _N, _N)],
                             adj_ref.at[8 * r + p], dsem)
            return c

        lax.fori_loop(0, 16, cout, 0)

        def cdrain(q, c):
            r = sid * 16 + q
            pltpu.make_async_copy(accum.at[pl.ds(r * _N, _N)],
                                  adj_ref.at[8 * r + p], dsem).wait()
            return c

        lax.fori_loop(0, 16, cdrain, 0)
        plsc.subcore_barrier()


def kernel(x, edge_index, batch_idx, B, N):
    adj0 = _adj_kernel(edge_index, batch_idx)
    dense_x = _dense_x(x, batch_idx)
    return (dense_x, adj0)


# speedup vs baseline: 1.0928x; 1.0928x over previous
import functools

import jax
import jax.numpy as jnp
from jax import lax
from jax.experimental import pallas as pl
from jax.experimental.pallas import tpu as pltpu
from jax.experimental.pallas import tpu_sc as plsc

_B = 8
_N = 2048
_TOTAL = 8192
_E = 262144
_D = 512
_BN = 512

_NTILE = 16
_EP = _E // _NTILE
_NG = _EP // 16
_NROWS = 128
_CHUNK = (_N // 8) * _N
_TRASH = _CHUNK
_HUGE = 1 << 30


def _dense_body(batch_ref, x_ref, out_ref, cum_smem, buf0, buf1, iota_v,
                sem0, sem1):
    b = pl.program_id(0)
    j = pl.program_id(1)
    nj = _N // _BN
    step = b * nj + j
    bufs = (buf0, buf1)
    sems = (sem0, sem1)
    L = _BN * _D

    def src_off(k):
        bb = k // nj
        jj = k % nj
        return (cum_smem[bb] + jj * _BN) * _D

    def start(k, buf, sem):
        pltpu.make_async_copy(x_ref.at[pl.ds(src_off(k), L)], buf, sem).start()

    def wait(k, buf, sem):
        pltpu.make_async_copy(x_ref.at[pl.ds(src_off(k), L)], buf, sem).wait()

    @pl.when(step == 0)
    def _():
        bi = batch_ref[...]
        for k in range(_B + 1):
            cum_smem[k] = jnp.sum((bi < k).astype(jnp.int32))
        start(0, buf0, sem0)
        start(1, buf1, sem1)
        iota_v[...] = lax.broadcasted_iota(jnp.int32, (L,), 0)

    parity = step % 2
    nsteps = _B * nj

    for par in range(2):
        @pl.when(parity == par)
        def _():
            buf, sem = bufs[par], sems[par]
            wait(step, buf, sem)
            thr = (cum_smem[b + 1] - cum_smem[b] - j * _BN) * _D

            @pl.when(thr >= L)
            def _():
                out_ref[...] = buf[...]

            @pl.when(thr <= 0)
            def _():
                out_ref[...] = jnp.zeros((L,), jnp.float32)

            @pl.when((thr > 0) & (thr < L))
            def _():
                out_ref[...] = jnp.where(iota_v[...] < thr, buf[...], 0.0)

            @pl.when(step + 2 < nsteps)
            def _():
                start(step + 2, buf, sem)


def _dense_x(x, batch_idx):
    x_pad = jnp.concatenate(
        [x.reshape(-1), jnp.zeros((_N * _D,), x.dtype)], axis=0)
    batch2d = batch_idx.reshape(_TOTAL // 128, 128)
    out = pl.pallas_call(
        _dense_body,
        grid=(_B, _N // _BN),
        in_specs=[
            pl.BlockSpec((_TOTAL // 128, 128), lambda b, j: (0, 0)),
            pl.BlockSpec(memory_space=pltpu.MemorySpace.HBM),
        ],
        out_specs=pl.BlockSpec((_BN * _D,), lambda b, j: (b * (_N // _BN) + j,)),
        out_shape=jax.ShapeDtypeStruct((_B * _N * _D,), jnp.float32),
        scratch_shapes=[
            pltpu.SMEM((_B + 1,), jnp.int32),
            pltpu.VMEM((_BN * _D,), jnp.float32),
            pltpu.VMEM((_BN * _D,), jnp.float32),
            pltpu.VMEM((_BN * _D,), jnp.int32),
            pltpu.SemaphoreType.DMA,
            pltpu.SemaphoreType.DMA,
        ],
    )(batch2d, x_pad)
    return out.reshape(_B, _N, _D)


_MESH = plsc.VectorSubcoreMesh(core_axis_name="c", subcore_axis_name="s")


_CROWS = 256
_CHUNKW = _CROWS * _N
_TRASHW = _CHUNKW


@functools.partial(
    pl.kernel,
    out_type=jax.ShapeDtypeStruct((_N, _N), jnp.float32),
    mesh=_MESH,
    compiler_params=pltpu.CompilerParams(needs_layout_passes=False),
    scratch_types=[
        pltpu.VMEM((_EP,), jnp.int32),
        pltpu.VMEM((_EP,), jnp.int32),
        pltpu.VMEM((_EP + 16,), jnp.int32),
        pltpu.VMEM((_TOTAL,), jnp.int32),
        pltpu.VMEM((16,), jnp.int32),
        pltpu.VMEM((256,), jnp.int32),
        pltpu.VMEM((_NROWS + 1, 128), jnp.int32),
        pltpu.VMEM((128,), jnp.float32),
        pltpu.VMEM((16384,), jnp.float32),
        pltpu.VMEM_SHARED((_CHUNKW + 64,), jnp.float32),
        pltpu.VMEM_SHARED((256,), jnp.int32),
        pltpu.SemaphoreType.DMA,
    ],
)
def _adj_kernel(edge_ref, batch_ref, adj_ref,
                src_v, dst_v, vlist, batch_v, cumtab, cumbuf, idx2d, ones_v,
                zeros_v, accum, cumstage, dsem):
    cid = lax.axis_index("c")
    sid = lax.axis_index("s")
    base_e = sid * _EP
    pltpu.async_copy(edge_ref.at[0, pl.ds(base_e, _EP)], src_v, dsem)
    pltpu.async_copy(edge_ref.at[1, pl.ds(base_e, _EP)], dst_v, dsem)
    pltpu.async_copy(batch_ref, batch_v, dsem)

    zero16f = jnp.zeros((16,), jnp.float32)

    def fill_z(i, c):
        zeros_v[pl.ds(i * 16, 16)] = zero16f
        return c

    lax.fori_loop(0, 16384 // 16, fill_z, 0)
    one16f = jnp.ones((16,), jnp.float32)
    for i in range(128 // 16):
        ones_v[pl.ds(i * 16, 16)] = one16f

    pltpu.make_async_copy(edge_ref.at[0, pl.ds(base_e, _EP)], src_v, dsem).wait()
    pltpu.make_async_copy(edge_ref.at[1, pl.ds(base_e, _EP)], dst_v, dsem).wait()
    pltpu.make_async_copy(batch_ref, batch_v, dsem).wait()

    one_b = jnp.full((16,), 1, jnp.int32)
    zero_b = jnp.full((16,), 0, jnp.int32)
    lanes = lax.broadcasted_iota(jnp.int32, (16,), 0)
    kconsts = [jnp.full((16,), k + 1, jnp.int32) for k in range(7)]

    gdn = lax.GatherDimensionNumbers(
        offset_dims=(), collapsed_slice_dims=(0,), start_index_map=(0,))

    def gat(v, idx):
        return lax.gather(v, idx[:, None], gdn, (1,),
                          mode=lax.GatherScatterMode.PROMISE_IN_BOUNDS)

    def lane_sum(v):
        for sh in (8, 4, 2, 1):
            v = v + gat(v, jnp.bitwise_xor(lanes, jnp.full((16,), sh,
                                                           jnp.int32)))
        return v

    def cum_body(g, accs):
        v = batch_v[pl.ds((sid * 32 + g) * 16, 16)]
        return tuple(a + jnp.where(v < kconsts[k], one_b, zero_b)
                     for k, a in enumerate(accs))

    accs = lax.fori_loop(0, 32, cum_body, tuple(zero_b for _ in range(7)))
    part = zero_b
    for k in range(7):
        part = jnp.where(lanes == kconsts[k], lane_sum(accs[k]), part)
    cumtab[pl.ds(0, 16)] = part
    pltpu.sync_copy(cumtab, cumstage.at[pl.ds(sid * 16, 16)])
    plsc.subcore_barrier()
    pltpu.sync_copy(cumstage, cumbuf)
    tot = zero_b
    for t in range(_NTILE):
        tot = tot + cumbuf[pl.ds(t * 16, 16)]
    cumtab[pl.ds(0, 16)] = tot

    n_b = jnp.full((16,), _N, jnp.int32)
    c1n = jnp.minimum(gat(tot, one_b), n_b)

    pidx = [jnp.maximum(lanes - jnp.full((16,), sh, jnp.int32), zero_b)
            for sh in (1, 2, 4, 8)]
    pmask = [lanes >= jnp.full((16,), sh, jnp.int32) for sh in (1, 2, 4, 8)]

    def prefix16(v):
        for i in range(4):
            v = v + jnp.where(pmask[i], gat(v, pidx[i]), zero_b)
        return v

    huge_b = jnp.full((16,), _HUGE, jnp.int32)
    c11_b = jnp.full((16,), 11, jnp.int32)
    m11_b = jnp.full((16,), 2047, jnp.int32)
    c7s_b = jnp.full((16,), 7, jnp.int32)
    m127_b = jnp.full((16,), 127, jnp.int32)
    c3s_b = jnp.full((16,), 3, jnp.int32)
    m7_b = jnp.full((16,), 7, jnp.int32)
    lim_b = jnp.full((16,), _CROWS, jnp.int32)
    trash_b = jnp.full((16,), _TRASHW, jnp.int32)

    def build(g, off_b):
        s = src_v[pl.ds(g * 16, 16)]
        t = dst_v[pl.ds(g * 16, 16)]
        bdst = plsc.load_gather(batch_v, [t])
        csel = plsc.load_gather(cumtab, [bdst])
        i2 = t - csel
        m = (s < c1n) & (i2 < n_b)
        gidx = lax.shift_left(s, c11_b) + i2
        m01 = jnp.where(m, one_b, zero_b)
        pref = prefix16(m01)
        pos = off_b + pref - m01
        plsc.store_scatter(vlist, [pos], gidx, mask=m)
        return off_b + plsc.all_reduce_population_count(m)

    off_b = lax.fori_loop(0, _NG, build, zero_b)
    nv = off_b[0]
    vlist[pl.ds(nv, 16)] = huge_b
    ngv = (nv + 15) // 16

    tile_zw = _CHUNKW // _NTILE

    for half in range(4):
        p = cid * 4 + half

        zslices = [(0, 16384), (16384, 16384)]
        for (zo, zl) in zslices:
            pltpu.async_copy(zeros_v.at[pl.ds(0, zl)],
                             accum.at[pl.ds(sid * tile_zw + zo, zl)], dsem)

        @pl.when(sid == 0)
        def _():
            pltpu.async_copy(zeros_v.at[pl.ds(0, 64)],
                             accum.at[pl.ds(_CHUNKW, 64)], dsem)

        pb = jnp.full((16,), p, jnp.int32)

        def mkloc(g, off2):
            gv = vlist[pl.ds(g * 16, 16)]
            i1 = lax.shift_right_logical(gv, c11_b)
            i2 = jnp.bitwise_and(gv, m11_b)
            r = lax.shift_right_logical(i1, c3s_b)
            m = (jnp.bitwise_and(i1, m7_b) == pb) & (r < lim_b)
            loc = jnp.bitwise_or(lax.shift_left(r, c11_b), i2)
            m01 = jnp.where(m, one_b, zero_b)
            pref = prefix16(m01)
            pos = off2 + pref - m01
            plsc.store_scatter(
                idx2d,
                [lax.shift_right_logical(pos, c7s_b),
                 jnp.bitwise_and(pos, m127_b)],
                loc, mask=m)
            return off2 + plsc.all_reduce_population_count(m)

        off2_b = lax.fori_loop(0, ngv, mkloc, zero_b)
        cnt2 = off2_b[0]

        for k in range(8):
            pos = off2_b + jnp.full((16,), k * 16, jnp.int32) + lanes
            plsc.store_scatter(
                idx2d,
                [lax.shift_right_logical(pos, c7s_b),
                 jnp.bitwise_and(pos, m127_b)],
                trash_b)

        ns = (cnt2 + 127) // 128

        for (zo, zl) in zslices:
            pltpu.make_async_copy(
                zeros_v.at[pl.ds(0, zl)],
                accum.at[pl.ds(sid * tile_zw + zo, zl)], dsem).wait()

        @pl.when(sid == 0)
        def _():
            pltpu.make_async_copy(zeros_v.at[pl.ds(0, 64)],
                                  accum.at[pl.ds(_CHUNKW, 64)], dsem).wait()

        plsc.subcore_barrier()

        def scat(j, c):
            pltpu.sync_copy(ones_v, accum.at[idx2d.at[j]], add=True)
            return c

        for t in range(_NTILE):
            @pl.when(sid == t)
            def _():
                lax.fori_loop(0, ns, scat, 0)

            plsc.subcore_barrier()
        plsc.subcore_barrier()

        def cout(q, c):
            r = q * 16 + sid
            pltpu.async_copy(accum.at[pl.ds(r * _N, _N)],
                             adj_ref.at[8 * r + p], dsem)
            return c

        lax.fori_loop(0, 16, cout, 0)

        def cdrain(q, c):
            r = q * 16 + sid
            pltpu.make_async_copy(accum.at[pl.ds(r * _N, _N)],
                                  adj_ref.at[8 * r + p], dsem).wait()
            return c

        lax.fori_loop(0, 16, cdrain, 0)
        plsc.subcore_barrier()


def kernel(x, edge_index, batch_idx, B, N):
    adj0 = _adj_kernel(edge_index, batch_idx)
    dense_x = _dense_x(x, batch_idx)
    return (dense_x, adj0)


# speedup vs baseline: 1.1749x; 1.0752x over previous
import functools

import jax
import jax.numpy as jnp
from jax import lax
from jax.experimental import pallas as pl
from jax.experimental.pallas import tpu as pltpu
from jax.experimental.pallas import tpu_sc as plsc

_B = 8
_N = 2048
_TOTAL = 8192
_E = 262144
_D = 512
_BN = 512

_NTILE = 16
_EP = _E // _NTILE
_NG = _EP // 16
_NROWS = 128
_CHUNK = (_N // 8) * _N
_TRASH = _CHUNK
_HUGE = 1 << 30


def _dense_body(batch_ref, x_ref, out_ref, cum_smem, buf0, buf1, iota_v,
                sem0, sem1):
    b = pl.program_id(0)
    j = pl.program_id(1)
    nj = _N // _BN
    step = b * nj + j
    bufs = (buf0, buf1)
    sems = (sem0, sem1)
    L = _BN * _D

    def src_off(k):
        bb = k // nj
        jj = k % nj
        return (cum_smem[bb] + jj * _BN) * _D

    def start(k, buf, sem):
        pltpu.make_async_copy(x_ref.at[pl.ds(src_off(k), L)], buf, sem).start()

    def wait(k, buf, sem):
        pltpu.make_async_copy(x_ref.at[pl.ds(src_off(k), L)], buf, sem).wait()

    @pl.when(step == 0)
    def _():
        bi = batch_ref[...]
        for k in range(_B + 1):
            cum_smem[k] = jnp.sum((bi < k).astype(jnp.int32))
        start(0, buf0, sem0)
        start(1, buf1, sem1)
        iota_v[...] = lax.broadcasted_iota(jnp.int32, (L,), 0)

    parity = step % 2
    nsteps = _B * nj

    for par in range(2):
        @pl.when(parity == par)
        def _():
            buf, sem = bufs[par], sems[par]
            wait(step, buf, sem)
            thr = (cum_smem[b + 1] - cum_smem[b] - j * _BN) * _D

            @pl.when(thr >= L)
            def _():
                out_ref[...] = buf[...]

            @pl.when(thr <= 0)
            def _():
                out_ref[...] = jnp.zeros((L,), jnp.float32)

            @pl.when((thr > 0) & (thr < L))
            def _():
                out_ref[...] = jnp.where(iota_v[...] < thr, buf[...], 0.0)

            @pl.when(step + 2 < nsteps)
            def _():
                start(step + 2, buf, sem)


def _dense_x(x, batch_idx):
    x_pad = jnp.concatenate(
        [x.reshape(-1), jnp.zeros((_N * _D,), x.dtype)], axis=0)
    batch2d = batch_idx.reshape(_TOTAL // 128, 128)
    out = pl.pallas_call(
        _dense_body,
        grid=(_B, _N // _BN),
        in_specs=[
            pl.BlockSpec((_TOTAL // 128, 128), lambda b, j: (0, 0)),
            pl.BlockSpec(memory_space=pltpu.MemorySpace.HBM),
        ],
        out_specs=pl.BlockSpec((_BN * _D,), lambda b, j: (b * (_N // _BN) + j,)),
        out_shape=jax.ShapeDtypeStruct((_B * _N * _D,), jnp.float32),
        scratch_shapes=[
            pltpu.SMEM((_B + 1,), jnp.int32),
            pltpu.VMEM((_BN * _D,), jnp.float32),
            pltpu.VMEM((_BN * _D,), jnp.float32),
            pltpu.VMEM((_BN * _D,), jnp.int32),
            pltpu.SemaphoreType.DMA,
            pltpu.SemaphoreType.DMA,
        ],
    )(batch2d, x_pad)
    return out.reshape(_B, _N, _D)


_MESH = plsc.VectorSubcoreMesh(core_axis_name="c", subcore_axis_name="s")


_CROWS = 256
_CHUNKW = _CROWS * _N
_TRASHW = _CHUNKW


@functools.partial(
    pl.kernel,
    out_type=jax.ShapeDtypeStruct((_N, _N), jnp.float32),
    mesh=_MESH,
    compiler_params=pltpu.CompilerParams(needs_layout_passes=False),
    scratch_types=[
        pltpu.VMEM((_EP,), jnp.int32),
        pltpu.VMEM((_EP,), jnp.int32),
        pltpu.VMEM((_EP + 16,), jnp.int32),
        pltpu.VMEM((_TOTAL,), jnp.int32),
        pltpu.VMEM((16,), jnp.int32),
        pltpu.VMEM((256,), jnp.int32),
        pltpu.VMEM((_NROWS + 1, 128), jnp.int32),
        pltpu.VMEM((128,), jnp.float32),
        pltpu.VMEM((16384,), jnp.float32),
        pltpu.VMEM_SHARED((_CHUNKW + 64,), jnp.float32),
        pltpu.VMEM_SHARED((256,), jnp.int32),
        pltpu.SemaphoreType.DMA,
    ],
)
def _adj_kernel(edge_ref, batch_ref, adj_ref,
                src_v, dst_v, vlist, batch_v, cumtab, cumbuf, idx2d, ones_v,
                zeros_v, accum, cumstage, dsem):
    cid = lax.axis_index("c")
    sid = lax.axis_index("s")
    base_e = sid * _EP
    pltpu.async_copy(edge_ref.at[0, pl.ds(base_e, _EP)], src_v, dsem)
    pltpu.async_copy(edge_ref.at[1, pl.ds(base_e, _EP)], dst_v, dsem)
    pltpu.async_copy(batch_ref, batch_v, dsem)

    zero16f = jnp.zeros((16,), jnp.float32)

    def fill_z(i, c):
        zeros_v[pl.ds(i * 16, 16)] = zero16f
        return c

    lax.fori_loop(0, 16384 // 16, fill_z, 0)
    one16f = jnp.ones((16,), jnp.float32)
    for i in range(128 // 16):
        ones_v[pl.ds(i * 16, 16)] = one16f

    pltpu.make_async_copy(edge_ref.at[0, pl.ds(base_e, _EP)], src_v, dsem).wait()
    pltpu.make_async_copy(edge_ref.at[1, pl.ds(base_e, _EP)], dst_v, dsem).wait()
    pltpu.make_async_copy(batch_ref, batch_v, dsem).wait()

    one_b = jnp.full((16,), 1, jnp.int32)
    zero_b = jnp.full((16,), 0, jnp.int32)
    lanes = lax.broadcasted_iota(jnp.int32, (16,), 0)
    kconsts = [jnp.full((16,), k + 1, jnp.int32) for k in range(7)]

    gdn = lax.GatherDimensionNumbers(
        offset_dims=(), collapsed_slice_dims=(0,), start_index_map=(0,))

    def gat(v, idx):
        return lax.gather(v, idx[:, None], gdn, (1,),
                          mode=lax.GatherScatterMode.PROMISE_IN_BOUNDS)

    def lane_sum(v):
        for sh in (8, 4, 2, 1):
            v = v + gat(v, jnp.bitwise_xor(lanes, jnp.full((16,), sh,
                                                           jnp.int32)))
        return v

    def cum_body(g, accs):
        v = batch_v[pl.ds((sid * 32 + g) * 16, 16)]
        return tuple(a + jnp.where(v < kconsts[k], one_b, zero_b)
                     for k, a in enumerate(accs))

    accs = lax.fori_loop(0, 32, cum_body, tuple(zero_b for _ in range(7)))
    part = zero_b
    for k in range(7):
        part = jnp.where(lanes == kconsts[k], lane_sum(accs[k]), part)
    cumtab[pl.ds(0, 16)] = part
    pltpu.sync_copy(cumtab, cumstage.at[pl.ds(sid * 16, 16)])
    plsc.subcore_barrier()
    pltpu.sync_copy(cumstage, cumbuf)
    tot = zero_b
    for t in range(_NTILE):
        tot = tot + cumbuf[pl.ds(t * 16, 16)]
    cumtab[pl.ds(0, 16)] = tot

    n_b = jnp.full((16,), _N, jnp.int32)
    c1n = jnp.minimum(gat(tot, one_b), n_b)

    pidx = [jnp.maximum(lanes - jnp.full((16,), sh, jnp.int32), zero_b)
            for sh in (1, 2, 4, 8)]
    pmask = [lanes >= jnp.full((16,), sh, jnp.int32) for sh in (1, 2, 4, 8)]

    def prefix16(v):
        for i in range(4):
            v = v + jnp.where(pmask[i], gat(v, pidx[i]), zero_b)
        return v

    huge_b = jnp.full((16,), _HUGE, jnp.int32)
    c11_b = jnp.full((16,), 11, jnp.int32)
    m11_b = jnp.full((16,), 2047, jnp.int32)
    c7s_b = jnp.full((16,), 7, jnp.int32)
    m127_b = jnp.full((16,), 127, jnp.int32)
    c3s_b = jnp.full((16,), 3, jnp.int32)
    m7_b = jnp.full((16,), 7, jnp.int32)
    lim_b = jnp.full((16,), _CROWS, jnp.int32)
    trash_b = jnp.full((16,), _TRASHW, jnp.int32)

    def build(g, off):
        s = src_v[pl.ds(g * 16, 16)]
        t = dst_v[pl.ds(g * 16, 16)]
        bdst = plsc.load_gather(batch_v, [t])
        csel = plsc.load_gather(cumtab, [bdst])
        i2 = t - csel
        m = (s < c1n) & (i2 < n_b)
        gidx = lax.shift_left(s, c11_b) + i2
        plsc.store_compressed(vlist.at[pl.ds(off, 16)], gidx, mask=m)
        pc = plsc.all_reduce_population_count(m)
        return off + pc[0]

    nv = lax.fori_loop(0, _NG, build, jnp.int32(0))
    vlist[pl.ds(nv, 16)] = huge_b
    ngv = (nv + 15) // 16

    tile_zw = _CHUNKW // _NTILE

    for half in range(4):
        p = cid * 4 + half

        zslices = [(0, 16384), (16384, 16384)]
        for (zo, zl) in zslices:
            pltpu.async_copy(zeros_v.at[pl.ds(0, zl)],
                             accum.at[pl.ds(sid * tile_zw + zo, zl)], dsem)

        @pl.when(sid == 0)
        def _():
            pltpu.async_copy(zeros_v.at[pl.ds(0, 64)],
                             accum.at[pl.ds(_CHUNKW, 64)], dsem)

        pb = jnp.full((16,), p, jnp.int32)

        def mkloc(g, off2):
            gv = vlist[pl.ds(g * 16, 16)]
            i1 = lax.shift_right_logical(gv, c11_b)
            i2 = jnp.bitwise_and(gv, m11_b)
            r = lax.shift_right_logical(i1, c3s_b)
            m = (jnp.bitwise_and(i1, m7_b) == pb) & (r < lim_b)
            loc = jnp.bitwise_or(lax.shift_left(r, c11_b), i2)
            m01 = jnp.where(m, one_b, zero_b)
            pref = prefix16(m01)
            pos = off2 + pref - m01
            plsc.store_scatter(
                idx2d,
                [lax.shift_right_logical(pos, c7s_b),
                 jnp.bitwise_and(pos, m127_b)],
                loc, mask=m)
            return off2 + plsc.all_reduce_population_count(m)

        off2_b = lax.fori_loop(0, ngv, mkloc, zero_b)
        cnt2 = off2_b[0]

        for k in range(8):
            pos = off2_b + jnp.full((16,), k * 16, jnp.int32) + lanes
            plsc.store_scatter(
                idx2d,
                [lax.shift_right_logical(pos, c7s_b),
                 jnp.bitwise_and(pos, m127_b)],
                trash_b)

        ns = (cnt2 + 127) // 128

        for (zo, zl) in zslices:
            pltpu.make_async_copy(
                zeros_v.at[pl.ds(0, zl)],
                accum.at[pl.ds(sid * tile_zw + zo, zl)], dsem).wait()

        @pl.when(sid == 0)
        def _():
            pltpu.make_async_copy(zeros_v.at[pl.ds(0, 64)],
                                  accum.at[pl.ds(_CHUNKW, 64)], dsem).wait()

        plsc.subcore_barrier()

        def scat(j, c):
            pltpu.sync_copy(ones_v, accum.at[idx2d.at[j]], add=True)
            return c

        for t in range(_NTILE):
            @pl.when(sid == t)
            def _():
                lax.fori_loop(0, ns, scat, 0)

            plsc.subcore_barrier()
        plsc.subcore_barrier()

        def cout(q, c):
            r = q * 16 + sid
            pltpu.async_copy(accum.at[pl.ds(r * _N, _N)],
                             adj_ref.at[8 * r + p], dsem)
            return c

        lax.fori_loop(0, 16, cout, 0)

        def cdrain(q, c):
            r = q * 16 + sid
            pltpu.make_async_copy(accum.at[pl.ds(r * _N, _N)],
                                  adj_ref.at[8 * r + p], dsem).wait()
            return c

        lax.fori_loop(0, 16, cdrain, 0)
        plsc.subcore_barrier()


def kernel(x, edge_index, batch_idx, B, N):
    adj0 = _adj_kernel(edge_index, batch_idx)
    dense_x = _dense_x(x, batch_idx)
    return (dense_x, adj0)


# speedup vs baseline: 1.1880x; 1.0111x over previous
import functools

import jax
import jax.numpy as jnp
from jax import lax
from jax.experimental import pallas as pl
from jax.experimental.pallas import tpu as pltpu
from jax.experimental.pallas import tpu_sc as plsc

_B = 8
_N = 2048
_TOTAL = 8192
_E = 262144
_D = 512
_BN = 512

_NTILE = 16
_EP = _E // _NTILE
_NG = _EP // 16
_NROWS = 128
_CHUNK = (_N // 8) * _N
_TRASH = _CHUNK
_HUGE = 1 << 30


def _dense_body(batch_ref, x_ref, out_ref, cum_smem, buf0, buf1, iota_v,
                sem0, sem1):
    b = pl.program_id(0)
    j = pl.program_id(1)
    nj = _N // _BN
    step = b * nj + j
    bufs = (buf0, buf1)
    sems = (sem0, sem1)
    L = _BN * _D
    o_max = _TOTAL * _D - L

    def src_off(k):
        bb = k // nj
        jj = k % nj
        o = (cum_smem[bb] + jj * _BN) * _D
        return jnp.minimum(o, o_max)

    def start(k, buf, sem):
        pltpu.make_async_copy(x_ref.at[pl.ds(src_off(k), L)], buf, sem).start()

    def wait(k, buf, sem):
        pltpu.make_async_copy(x_ref.at[pl.ds(src_off(k), L)], buf, sem).wait()

    @pl.when(step == 0)
    def _():
        bi = batch_ref[...]
        for k in range(_B + 1):
            cum_smem[k] = jnp.sum((bi < k).astype(jnp.int32))
        start(0, buf0, sem0)
        start(1, buf1, sem1)
        iota_v[...] = lax.broadcasted_iota(jnp.int32, (L,), 0)

    parity = step % 2
    nsteps = _B * nj

    for par in range(2):
        @pl.when(parity == par)
        def _():
            buf, sem = bufs[par], sems[par]
            wait(step, buf, sem)
            o = (cum_smem[b] + j * _BN) * _D
            dlt = o - jnp.minimum(o, o_max)
            thr = (cum_smem[b + 1] - cum_smem[b] - j * _BN) * _D

            @pl.when((thr >= L) & (dlt == 0))
            def _():
                out_ref[...] = buf[...]

            @pl.when(thr <= 0)
            def _():
                out_ref[...] = jnp.zeros((L,), jnp.float32)

            @pl.when((thr > 0) & (thr < L) & (dlt == 0))
            def _():
                out_ref[...] = jnp.where(iota_v[...] < thr, buf[...], 0.0)

            @pl.when((thr > 0) & (dlt > 0))
            def _():
                nr = thr // _D
                nck = nr // 64
                nrem = nr - nck * 64
                ck = 64 * _D

                def fire_ck(k, c):
                    pltpu.make_async_copy(
                        x_ref.at[pl.ds(o + k * ck, ck)],
                        buf.at[pl.ds(k * ck, ck)], sem).start()
                    return c

                def fire_r(r2, c):
                    pltpu.make_async_copy(
                        x_ref.at[pl.ds(o + nck * ck + r2 * _D, _D)],
                        buf.at[pl.ds(nck * ck + r2 * _D, _D)], sem).start()
                    return c

                def drain_ck(k, c):
                    pltpu.make_async_copy(
                        x_ref.at[pl.ds(o + k * ck, ck)],
                        buf.at[pl.ds(k * ck, ck)], sem).wait()
                    return c

                def drain_r(r2, c):
                    pltpu.make_async_copy(
                        x_ref.at[pl.ds(o + nck * ck + r2 * _D, _D)],
                        buf.at[pl.ds(nck * ck + r2 * _D, _D)], sem).wait()
                    return c

                lax.fori_loop(0, nck, fire_ck, 0)
                lax.fori_loop(0, nrem, fire_r, 0)
                lax.fori_loop(0, nck, drain_ck, 0)
                lax.fori_loop(0, nrem, drain_r, 0)
                out_ref[...] = jnp.where(iota_v[...] < thr, buf[...], 0.0)

            @pl.when(step + 2 < nsteps)
            def _():
                start(step + 2, buf, sem)


def _dense_x(x, batch_idx):
    batch2d = batch_idx.reshape(_TOTAL // 128, 128)
    out = pl.pallas_call(
        _dense_body,
        grid=(_B, _N // _BN),
        in_specs=[
            pl.BlockSpec((_TOTAL // 128, 128), lambda b, j: (0, 0)),
            pl.BlockSpec(memory_space=pltpu.MemorySpace.HBM),
        ],
        out_specs=pl.BlockSpec((_BN * _D,), lambda b, j: (b * (_N // _BN) + j,)),
        out_shape=jax.ShapeDtypeStruct((_B * _N * _D,), jnp.float32),
        scratch_shapes=[
            pltpu.SMEM((_B + 1,), jnp.int32),
            pltpu.VMEM((_BN * _D,), jnp.float32),
            pltpu.VMEM((_BN * _D,), jnp.float32),
            pltpu.VMEM((_BN * _D,), jnp.int32),
            pltpu.SemaphoreType.DMA,
            pltpu.SemaphoreType.DMA,
        ],
    )(batch2d, x.reshape(-1))
    return out.reshape(_B, _N, _D)


_MESH = plsc.VectorSubcoreMesh(core_axis_name="c", subcore_axis_name="s")


_CROWS = 256
_CHUNKW = _CROWS * _N
_TRASHW = _CHUNKW


@functools.partial(
    pl.kernel,
    out_type=jax.ShapeDtypeStruct((_N, _N), jnp.float32),
    mesh=_MESH,
    compiler_params=pltpu.CompilerParams(needs_layout_passes=False),
    scratch_types=[
        pltpu.VMEM((_EP,), jnp.int32),
        pltpu.VMEM((_EP,), jnp.int32),
        pltpu.VMEM((_EP + 16,), jnp.int32),
        pltpu.VMEM((_TOTAL,), jnp.int32),
        pltpu.VMEM((16,), jnp.int32),
        pltpu.VMEM((256,), jnp.int32),
        pltpu.VMEM((_NROWS + 1, 128), jnp.int32),
        pltpu.VMEM((128,), jnp.float32),
        pltpu.VMEM((16384,), jnp.float32),
        pltpu.VMEM_SHARED((_CHUNKW + 64,), jnp.float32),
        pltpu.VMEM_SHARED((256,), jnp.int32),
        pltpu.SemaphoreType.DMA,
    ],
)
def _adj_kernel(edge_ref, batch_ref, adj_ref,
                src_v, dst_v, vlist, batch_v, cumtab, cumbuf, idx2d, ones_v,
                zeros_v, accum, cumstage, dsem):
    cid = lax.axis_index("c")
    sid = lax.axis_index("s")
    base_e = sid * _EP
    pltpu.async_copy(edge_ref.at[0, pl.ds(base_e, _EP)], src_v, dsem)
    pltpu.async_copy(edge_ref.at[1, pl.ds(base_e, _EP)], dst_v, dsem)
    pltpu.async_copy(batch_ref, batch_v, dsem)

    zero16f = jnp.zeros((16,), jnp.float32)

    def fill_z(i, c):
        zeros_v[pl.ds(i * 16, 16)] = zero16f
        return c

    lax.fori_loop(0, 16384 // 16, fill_z, 0)
    one16f = jnp.ones((16,), jnp.float32)
    for i in range(128 // 16):
        ones_v[pl.ds(i * 16, 16)] = one16f

    pltpu.make_async_copy(edge_ref.at[0, pl.ds(base_e, _EP)], src_v, dsem).wait()
    pltpu.make_async_copy(edge_ref.at[1, pl.ds(base_e, _EP)], dst_v, dsem).wait()
    pltpu.make_async_copy(batch_ref, batch_v, dsem).wait()

    one_b = jnp.full((16,), 1, jnp.int32)
    zero_b = jnp.full((16,), 0, jnp.int32)
    lanes = lax.broadcasted_iota(jnp.int32, (16,), 0)
    kconsts = [jnp.full((16,), k + 1, jnp.int32) for k in range(7)]

    gdn = lax.GatherDimensionNumbers(
        offset_dims=(), collapsed_slice_dims=(0,), start_index_map=(0,))

    def gat(v, idx):
        return lax.gather(v, idx[:, None], gdn, (1,),
                          mode=lax.GatherScatterMode.PROMISE_IN_BOUNDS)

    def lane_sum(v):
        for sh in (8, 4, 2, 1):
            v = v + gat(v, jnp.bitwise_xor(lanes, jnp.full((16,), sh,
                                                           jnp.int32)))
        return v

    def cum_body(g, accs):
        v = batch_v[pl.ds((sid * 32 + g) * 16, 16)]
        return tuple(a + jnp.where(v < kconsts[k], one_b, zero_b)
                     for k, a in enumerate(accs))

    accs = lax.fori_loop(0, 32, cum_body, tuple(zero_b for _ in range(7)))
    part = zero_b
    for k in range(7):
        part = jnp.where(lanes == kconsts[k], lane_sum(accs[k]), part)
    cumtab[pl.ds(0, 16)] = part
    pltpu.sync_copy(cumtab, cumstage.at[pl.ds(sid * 16, 16)])
    plsc.subcore_barrier()
    pltpu.sync_copy(cumstage, cumbuf)
    tot = zero_b
    for t in range(_NTILE):
        tot = tot + cumbuf[pl.ds(t * 16, 16)]
    cumtab[pl.ds(0, 16)] = tot

    n_b = jnp.full((16,), _N, jnp.int32)
    c1n = jnp.minimum(gat(tot, one_b), n_b)

    pidx = [jnp.maximum(lanes - jnp.full((16,), sh, jnp.int32), zero_b)
            for sh in (1, 2, 4, 8)]
    pmask = [lanes >= jnp.full((16,), sh, jnp.int32) for sh in (1, 2, 4, 8)]

    def prefix16(v):
        for i in range(4):
            v = v + jnp.where(pmask[i], gat(v, pidx[i]), zero_b)
        return v

    huge_b = jnp.full((16,), _HUGE, jnp.int32)
    c11_b = jnp.full((16,), 11, jnp.int32)
    m11_b = jnp.full((16,), 2047, jnp.int32)
    c7s_b = jnp.full((16,), 7, jnp.int32)
    m127_b = jnp.full((16,), 127, jnp.int32)
    c3s_b = jnp.full((16,), 3, jnp.int32)
    m7_b = jnp.full((16,), 7, jnp.int32)
    lim_b = jnp.full((16,), _CROWS, jnp.int32)
    trash_b = jnp.full((16,), _TRASHW, jnp.int32)

    def build(g, off):
        s = src_v[pl.ds(g * 16, 16)]
        t = dst_v[pl.ds(g * 16, 16)]
        bdst = plsc.load_gather(batch_v, [t])
        csel = plsc.load_gather(cumtab, [bdst])
        i2 = t - csel
        m = (s < c1n) & (i2 < n_b)
        gidx = lax.shift_left(s, c11_b) + i2
        plsc.store_compressed(vlist.at[pl.ds(off, 16)], gidx, mask=m)
        pc = plsc.all_reduce_population_count(m)
        return off + pc[0]

    nv = lax.fori_loop(0, _NG, build, jnp.int32(0))
    vlist[pl.ds(nv, 16)] = huge_b
    ngv = (nv + 15) // 16

    tile_zw = _CHUNKW // _NTILE

    for half in range(4):
        p = cid * 4 + half

        zslices = [(0, 16384), (16384, 16384)]
        for (zo, zl) in zslices:
            pltpu.async_copy(zeros_v.at[pl.ds(0, zl)],
                             accum.at[pl.ds(sid * tile_zw + zo, zl)], dsem)

        @pl.when(sid == 0)
        def _():
            pltpu.async_copy(zeros_v.at[pl.ds(0, 64)],
                             accum.at[pl.ds(_CHUNKW, 64)], dsem)

        pb = jnp.full((16,), p, jnp.int32)

        def mkloc(g, off2):
            gv = vlist[pl.ds(g * 16, 16)]
            i1 = lax.shift_right_logical(gv, c11_b)
            i2 = jnp.bitwise_and(gv, m11_b)
            r = lax.shift_right_logical(i1, c3s_b)
            m = (jnp.bitwise_and(i1, m7_b) == pb) & (r < lim_b)
            loc = jnp.bitwise_or(lax.shift_left(r, c11_b), i2)
            m01 = jnp.where(m, one_b, zero_b)
            pref = prefix16(m01)
            pos = off2 + pref - m01
            plsc.store_scatter(
                idx2d,
                [lax.shift_right_logical(pos, c7s_b),
                 jnp.bitwise_and(pos, m127_b)],
                loc, mask=m)
            return off2 + plsc.all_reduce_population_count(m)

        off2_b = lax.fori_loop(0, ngv, mkloc, zero_b)
        cnt2 = off2_b[0]

        for k in range(8):
            pos = off2_b + jnp.full((16,), k * 16, jnp.int32) + lanes
            plsc.store_scatter(
                idx2d,
                [lax.shift_right_logical(pos, c7s_b),
                 jnp.bitwise_and(pos, m127_b)],
                trash_b)

        ns = (cnt2 + 127) // 128

        for (zo, zl) in zslices:
            pltpu.make_async_copy(
                zeros_v.at[pl.ds(0, zl)],
                accum.at[pl.ds(sid * tile_zw + zo, zl)], dsem).wait()

        @pl.when(sid == 0)
        def _():
            pltpu.make_async_copy(zeros_v.at[pl.ds(0, 64)],
                                  accum.at[pl.ds(_CHUNKW, 64)], dsem).wait()

        plsc.subcore_barrier()

        def scat(j, c):
            pltpu.sync_copy(ones_v, accum.at[idx2d.at[j]], add=True)
            return c

        for t in range(_NTILE):
            @pl.when(sid == t)
            def _():
                lax.fori_loop(0, ns, scat, 0)

            plsc.subcore_barrier()
        plsc.subcore_barrier()

        def cout(q, c):
            r = q * 16 + sid
            pltpu.async_copy(accum.at[pl.ds(r * _N, _N)],
                             adj_ref.at[8 * r + p], dsem)
            return c

        lax.fori_loop(0, 16, cout, 0)

        def cdrain(q, c):
            r = q * 16 + sid
            pltpu.make_async_copy(accum.at[pl.ds(r * _N, _N)],
                                  adj_ref.at[8 * r + p], dsem).wait()
            return c

        lax.fori_loop(0, 16, cdrain, 0)
        plsc.subcore_barrier()


def kernel(x, edge_index, batch_idx, B, N):
    adj0 = _adj_kernel(edge_index, batch_idx)
    dense_x = _dense_x(x, batch_idx)
    return (dense_x, adj0)


# speedup vs baseline: 1.2509x; 1.0529x over previous
import functools

import jax
import jax.numpy as jnp
from jax import lax
from jax.experimental import pallas as pl
from jax.experimental.pallas import tpu as pltpu
from jax.experimental.pallas import tpu_sc as plsc

_B = 8
_N = 2048
_TOTAL = 8192
_E = 262144
_D = 512
_BN = 512

_NTILE = 16
_EP = _E // _NTILE
_NG = _EP // 16
_NROWS = 128
_CHUNK = (_N // 8) * _N
_TRASH = _CHUNK
_HUGE = 1 << 30


def _dense_body(batch_ref, x_ref, out_ref, cum_smem, buf0, buf1, iota_v,
                sem0, sem1):
    b = pl.program_id(0)
    j = pl.program_id(1)
    nj = _N // _BN
    step = b * nj + j
    bufs = (buf0, buf1)
    sems = (sem0, sem1)
    L = _BN * _D
    o_max = _TOTAL * _D - L

    def src_off(k):
        bb = k // nj
        jj = k % nj
        o = (cum_smem[bb] + jj * _BN) * _D
        return jnp.minimum(o, o_max)

    def start(k, buf, sem):
        pltpu.make_async_copy(x_ref.at[pl.ds(src_off(k), L)], buf, sem).start()

    def wait(k, buf, sem):
        pltpu.make_async_copy(x_ref.at[pl.ds(src_off(k), L)], buf, sem).wait()

    @pl.when(step == 0)
    def _():
        bi = batch_ref[...]
        for k in range(_B + 1):
            cum_smem[k] = jnp.sum((bi < k).astype(jnp.int32))
        start(0, buf0, sem0)
        start(1, buf1, sem1)
        iota_v[...] = lax.broadcasted_iota(jnp.int32, (L,), 0)

    parity = step % 2
    nsteps = _B * nj

    for par in range(2):
        @pl.when(parity == par)
        def _():
            buf, sem = bufs[par], sems[par]
            wait(step, buf, sem)
            o = (cum_smem[b] + j * _BN) * _D
            dlt = o - jnp.minimum(o, o_max)
            thr = (cum_smem[b + 1] - cum_smem[b] - j * _BN) * _D

            @pl.when((thr >= L) & (dlt == 0))
            def _():
                out_ref[...] = buf[...]

            @pl.when(thr <= 0)
            def _():
                out_ref[...] = jnp.zeros((L,), jnp.float32)

            @pl.when((thr > 0) & (thr < L) & (dlt == 0))
            def _():
                out_ref[...] = jnp.where(iota_v[...] < thr, buf[...], 0.0)

            @pl.when((thr > 0) & (dlt > 0))
            def _():
                nr = thr // _D
                nck = nr // 64
                nrem = nr - nck * 64
                ck = 64 * _D

                def fire_ck(k, c):
                    pltpu.make_async_copy(
                        x_ref.at[pl.ds(o + k * ck, ck)],
                        buf.at[pl.ds(k * ck, ck)], sem).start()
                    return c

                def fire_r(r2, c):
                    pltpu.make_async_copy(
                        x_ref.at[pl.ds(o + nck * ck + r2 * _D, _D)],
                        buf.at[pl.ds(nck * ck + r2 * _D, _D)], sem).start()
                    return c

                def drain_ck(k, c):
                    pltpu.make_async_copy(
                        x_ref.at[pl.ds(o + k * ck, ck)],
                        buf.at[pl.ds(k * ck, ck)], sem).wait()
                    return c

                def drain_r(r2, c):
                    pltpu.make_async_copy(
                        x_ref.at[pl.ds(o + nck * ck + r2 * _D, _D)],
                        buf.at[pl.ds(nck * ck + r2 * _D, _D)], sem).wait()
                    return c

                lax.fori_loop(0, nck, fire_ck, 0)
                lax.fori_loop(0, nrem, fire_r, 0)
                lax.fori_loop(0, nck, drain_ck, 0)
                lax.fori_loop(0, nrem, drain_r, 0)
                out_ref[...] = jnp.where(iota_v[...] < thr, buf[...], 0.0)

            @pl.when(step + 2 < nsteps)
            def _():
                start(step + 2, buf, sem)


def _dense_x(x, batch_idx):
    batch2d = batch_idx.reshape(_TOTAL // 128, 128)
    out = pl.pallas_call(
        _dense_body,
        grid=(_B, _N // _BN),
        in_specs=[
            pl.BlockSpec((_TOTAL // 128, 128), lambda b, j: (0, 0)),
            pl.BlockSpec(memory_space=pltpu.MemorySpace.HBM),
        ],
        out_specs=pl.BlockSpec((_BN * _D,), lambda b, j: (b * (_N // _BN) + j,)),
        out_shape=jax.ShapeDtypeStruct((_B * _N * _D,), jnp.float32),
        scratch_shapes=[
            pltpu.SMEM((_B + 1,), jnp.int32),
            pltpu.VMEM((_BN * _D,), jnp.float32),
            pltpu.VMEM((_BN * _D,), jnp.float32),
            pltpu.VMEM((_BN * _D,), jnp.int32),
            pltpu.SemaphoreType.DMA,
            pltpu.SemaphoreType.DMA,
        ],
    )(batch2d, x.reshape(-1))
    return out.reshape(_B, _N, _D)


_MESH = plsc.VectorSubcoreMesh(core_axis_name="c", subcore_axis_name="s")


_CROWS = 256
_CHUNKW = _CROWS * _N
_TRASHW = _CHUNKW


@functools.partial(
    pl.kernel,
    out_type=jax.ShapeDtypeStruct((_N, _N), jnp.float32),
    mesh=_MESH,
    compiler_params=pltpu.CompilerParams(needs_layout_passes=False),
    scratch_types=[
        pltpu.VMEM((_EP // 2,), jnp.int32),
        pltpu.VMEM((_EP // 2,), jnp.int32),
        pltpu.VMEM((_EP + 16,), jnp.int32),
        pltpu.VMEM((_TOTAL,), jnp.int32),
        pltpu.VMEM((16,), jnp.int32),
        pltpu.VMEM((256,), jnp.int32),
        pltpu.VMEM((_NROWS + 1, 128), jnp.int32),
        pltpu.VMEM((_NROWS + 1, 128), jnp.int32),
        pltpu.VMEM((128,), jnp.float32),
        pltpu.VMEM((8192,), jnp.float32),
        pltpu.VMEM_SHARED((_CHUNKW + 64,), jnp.float32),
        pltpu.VMEM_SHARED((256,), jnp.int32),
        pltpu.SemaphoreType.DMA,
    ],
)
def _adj_kernel(edge_ref, batch_ref, adj_ref,
                src_v, dst_v, vlist, batch_v, cumtab, cumbuf, idx2d, idx2db,
                ones_v, zeros_v, accum, cumstage, dsem):
    cid = lax.axis_index("c")
    sid = lax.axis_index("s")
    base_e = sid * _EP
    pltpu.async_copy(edge_ref.at[0, pl.ds(base_e, _EP // 2)], src_v, dsem)
    pltpu.async_copy(edge_ref.at[1, pl.ds(base_e, _EP // 2)], dst_v, dsem)
    pltpu.async_copy(batch_ref, batch_v, dsem)

    zero16f = jnp.zeros((16,), jnp.float32)

    def fill_z(i, c):
        zeros_v[pl.ds(i * 16, 16)] = zero16f
        return c

    lax.fori_loop(0, 8192 // 16, fill_z, 0)
    one16f = jnp.ones((16,), jnp.float32)
    for i in range(128 // 16):
        ones_v[pl.ds(i * 16, 16)] = one16f

    pltpu.make_async_copy(edge_ref.at[0, pl.ds(base_e, _EP // 2)], src_v,
                          dsem).wait()
    pltpu.make_async_copy(edge_ref.at[1, pl.ds(base_e, _EP // 2)], dst_v,
                          dsem).wait()
    pltpu.make_async_copy(batch_ref, batch_v, dsem).wait()

    one_b = jnp.full((16,), 1, jnp.int32)
    zero_b = jnp.full((16,), 0, jnp.int32)
    lanes = lax.broadcasted_iota(jnp.int32, (16,), 0)
    kconsts = [jnp.full((16,), k + 1, jnp.int32) for k in range(7)]

    gdn = lax.GatherDimensionNumbers(
        offset_dims=(), collapsed_slice_dims=(0,), start_index_map=(0,))

    def gat(v, idx):
        return lax.gather(v, idx[:, None], gdn, (1,),
                          mode=lax.GatherScatterMode.PROMISE_IN_BOUNDS)

    def lane_sum(v):
        for sh in (8, 4, 2, 1):
            v = v + gat(v, jnp.bitwise_xor(lanes, jnp.full((16,), sh,
                                                           jnp.int32)))
        return v

    def cum_body(g, accs):
        v = batch_v[pl.ds((sid * 32 + g) * 16, 16)]
        return tuple(a + jnp.where(v < kconsts[k], one_b, zero_b)
                     for k, a in enumerate(accs))

    accs = lax.fori_loop(0, 32, cum_body, tuple(zero_b for _ in range(7)))
    part = zero_b
    for k in range(7):
        part = jnp.where(lanes == kconsts[k], lane_sum(accs[k]), part)
    cumtab[pl.ds(0, 16)] = part
    pltpu.sync_copy(cumtab, cumstage.at[pl.ds(sid * 16, 16)])
    plsc.subcore_barrier()
    pltpu.sync_copy(cumstage, cumbuf)
    tot = zero_b
    for t in range(_NTILE):
        tot = tot + cumbuf[pl.ds(t * 16, 16)]
    cumtab[pl.ds(0, 16)] = tot

    n_b = jnp.full((16,), _N, jnp.int32)
    c1n = jnp.minimum(gat(tot, one_b), n_b)

    pidx = [jnp.maximum(lanes - jnp.full((16,), sh, jnp.int32), zero_b)
            for sh in (1, 2, 4, 8)]
    pmask = [lanes >= jnp.full((16,), sh, jnp.int32) for sh in (1, 2, 4, 8)]

    def prefix16(v):
        for i in range(4):
            v = v + jnp.where(pmask[i], gat(v, pidx[i]), zero_b)
        return v

    huge_b = jnp.full((16,), _HUGE, jnp.int32)
    c11_b = jnp.full((16,), 11, jnp.int32)
    m11_b = jnp.full((16,), 2047, jnp.int32)
    c7s_b = jnp.full((16,), 7, jnp.int32)
    m127_b = jnp.full((16,), 127, jnp.int32)
    c3s_b = jnp.full((16,), 3, jnp.int32)
    m7_b = jnp.full((16,), 7, jnp.int32)
    lim_b = jnp.full((16,), _CROWS, jnp.int32)
    trash_b = jnp.full((16,), _TRASHW, jnp.int32)

    def build(g, off):
        s = src_v[pl.ds(g * 16, 16)]
        t = dst_v[pl.ds(g * 16, 16)]
        bdst = plsc.load_gather(batch_v, [t])
        csel = plsc.load_gather(cumtab, [bdst])
        i2 = t - csel
        m = (s < c1n) & (i2 < n_b)
        gidx = lax.shift_left(s, c11_b) + i2
        plsc.store_compressed(vlist.at[pl.ds(off, 16)], gidx, mask=m)
        pc = plsc.all_reduce_population_count(m)
        return off + pc[0]

    nv = lax.fori_loop(0, _NG // 2, build, jnp.int32(0))
    pltpu.sync_copy(edge_ref.at[0, pl.ds(base_e + _EP // 2, _EP // 2)], src_v)
    pltpu.sync_copy(edge_ref.at[1, pl.ds(base_e + _EP // 2, _EP // 2)], dst_v)
    nv = lax.fori_loop(0, _NG // 2, build, nv)
    vlist[pl.ds(nv, 16)] = huge_b
    ngv = (nv + 15) // 16

    tile_zw = _CHUNKW // _NTILE

    for half in range(4):
        p = cid * 4 + half

        zslices = [(0, 8192), (8192, 8192), (16384, 8192), (24576, 8192)]
        for (zo, zl) in zslices:
            pltpu.async_copy(zeros_v.at[pl.ds(0, zl)],
                             accum.at[pl.ds(sid * tile_zw + zo, zl)], dsem)

        @pl.when(sid == 0)
        def _():
            pltpu.async_copy(zeros_v.at[pl.ds(0, 64)],
                             accum.at[pl.ds(_CHUNKW, 64)], dsem)

        pb = jnp.full((16,), p, jnp.int32)

        def mkloc(g, offs):
            offa, offb = offs
            gv = vlist[pl.ds(g * 16, 16)]
            i1 = lax.shift_right_logical(gv, c11_b)
            i2 = jnp.bitwise_and(gv, m11_b)
            r = lax.shift_right_logical(i1, c3s_b)
            m = (jnp.bitwise_and(i1, m7_b) == pb) & (r < lim_b)
            loc = jnp.bitwise_or(lax.shift_left(r, c11_b), i2)
            par = jnp.bitwise_and(loc, one_b)
            ma = m & (par == zero_b)
            mb = m & (par == one_b)
            for (buf2d, mm, off) in ((idx2d, ma, offa), (idx2db, mb, offb)):
                m01 = jnp.where(mm, one_b, zero_b)
                pos = off + prefix16(m01) - m01
                plsc.store_scatter(
                    buf2d,
                    [lax.shift_right_logical(pos, c7s_b),
                     jnp.bitwise_and(pos, m127_b)],
                    loc, mask=mm)
            return (offa + plsc.all_reduce_population_count(ma),
                    offb + plsc.all_reduce_population_count(mb))

        offa_b, offb_b = lax.fori_loop(0, ngv, mkloc, (zero_b, zero_b))

        for (buf2d, off) in ((idx2d, offa_b), (idx2db, offb_b)):
            for k in range(8):
                pos = off + jnp.full((16,), k * 16, jnp.int32) + lanes
                plsc.store_scatter(
                    buf2d,
                    [lax.shift_right_logical(pos, c7s_b),
                     jnp.bitwise_and(pos, m127_b)],
                    trash_b)

        nsa = (offa_b[0] + 127) // 128
        nsb = (offb_b[0] + 127) // 128

        for (zo, zl) in zslices:
            pltpu.make_async_copy(
                zeros_v.at[pl.ds(0, zl)],
                accum.at[pl.ds(sid * tile_zw + zo, zl)], dsem).wait()

        @pl.when(sid == 0)
        def _():
            pltpu.make_async_copy(zeros_v.at[pl.ds(0, 64)],
                                  accum.at[pl.ds(_CHUNKW, 64)], dsem).wait()

        plsc.subcore_barrier()

        def scata(j, c):
            pltpu.sync_copy(ones_v, accum.at[idx2d.at[j]], add=True)
            return c

        def scatb(j, c):
            pltpu.sync_copy(ones_v, accum.at[idx2db.at[j]], add=True)
            return c

        def round_body(t, c):
            @pl.when(sid == t)
            def _():
                lax.fori_loop(0, nsa, scata, 0)

            @pl.when(sid == t + 8)
            def _():
                lax.fori_loop(0, nsb, scatb, 0)

            plsc.subcore_barrier()

            @pl.when(sid == t)
            def _():
                lax.fori_loop(0, nsb, scatb, 0)

            @pl.when(sid == t + 8)
            def _():
                lax.fori_loop(0, nsa, scata, 0)

            plsc.subcore_barrier()
            return c

        lax.fori_loop(0, 8, round_body, 0)
        plsc.subcore_barrier()

        def cout(q, c):
            r = q * 16 + sid
            pltpu.async_copy(accum.at[pl.ds(r * _N, _N)],
                             adj_ref.at[8 * r + p], dsem)
            return c

        lax.fori_loop(0, 16, cout, 0)

        def cdrain(q, c):
            r = q * 16 + sid
            pltpu.make_async_copy(accum.at[pl.ds(r * _N, _N)],
                                  adj_ref.at[8 * r + p], dsem).wait()
            return c

        lax.fori_loop(0, 16, cdrain, 0)
        plsc.subcore_barrier()


def kernel(x, edge_index, batch_idx, B, N):
    adj0 = _adj_kernel(edge_index, batch_idx)
    dense_x = _dense_x(x, batch_idx)
    return (dense_x, adj0)
